# baseline (speedup 1.0000x reference)
import functools

import numpy as np
import jax
import jax.numpy as jnp
from jax import lax
from jax.experimental import pallas as pl
from jax.experimental.pallas import tpu as pltpu
from jax.experimental.pallas import tpu_sc as plsc

_N, _H, _W, _C = 4, 384, 384, 96
_OH = _OW = 224
_CH = _C // 2
_NTASK = _N * _OH * 2
_NWORK = 32
_TPW = _NTASK // _NWORK
_NJC = _OW // 16


def _axis_tables():
    scale = np.float32(_W / _OW)
    i = np.arange(_OW, dtype=np.float32)
    src = (i + np.float32(0.5)) * scale - np.float32(0.5)
    src = np.maximum(src, np.float32(0.0))
    lo = np.floor(src).astype(np.int32)
    frac = (src - lo.astype(np.float32)).astype(np.float32)
    return lo, frac


def _tables():
    lo, frac = _axis_tables()
    x0_t = lo.copy()
    fx_t = frac.copy()
    t = np.arange(_NTASK, dtype=np.int32)
    i_out = (t // 2) % _OH
    fy_t = np.broadcast_to(frac[i_out][:, None], (_NTASK, 16)).astype(np.float32).reshape(-1).copy()
    return x0_t, fx_t, fy_t


_X0, _FX, _FY = _tables()


def _sc_resize(img3):
    mesh = plsc.VectorSubcoreMesh(core_axis_name="c", subcore_axis_name="s")

    @functools.partial(
        pl.kernel,
        out_type=jax.ShapeDtypeStruct((_NTASK, _CH, _OW), jnp.float32),
        mesh=mesh,
        scratch_types=[
            pltpu.VMEM((_OW,), jnp.int32),
            pltpu.VMEM((_OW,), jnp.float32),
            pltpu.VMEM((_TPW * 16,), jnp.float32),
            pltpu.VMEM((4 * _CH, _W), jnp.float32),
            pltpu.VMEM((_CH, _OW), jnp.float32),
            pltpu.SemaphoreType.DMA,
            pltpu.SemaphoreType.DMA,
        ],
        compiler_params=pltpu.CompilerParams(needs_layout_passes=False),
    )
    def run(img_hbm, x0_hbm, fx_hbm, fy_hbm, out_hbm,
            x0_v, fx_v, fy_v, rows_v, out_v, rsem0, rsem1):
        wid = lax.axis_index("s") * 2 + lax.axis_index("c")
        base = wid * _TPW
        pltpu.sync_copy(x0_hbm, x0_v)
        pltpu.sync_copy(fx_hbm, fx_v)
        pltpu.sync_copy(fy_hbm.at[pl.ds(base * 16, _TPW * 16)], fy_v)

        rsems = (rsem0, rsem1)
        zlane = lax.iota(jnp.int32, 16) * 0

        def row_copies(k, slot):
            t = base + k
            ch = lax.rem(t, 2)
            i = lax.rem(lax.div(t, 2), _OH)
            n = lax.div(t, 2 * _OH)
            y0 = lax.div(24 * i + 5, 14)
            s0 = (n * _H + y0) * 2 + ch
            sem = rsems[slot]
            off = slot * 2 * _CH
            return (
                pltpu.make_async_copy(
                    img_hbm.at[s0], rows_v.at[pl.ds(off, _CH)], sem),
                pltpu.make_async_copy(
                    img_hbm.at[s0 + 2],
                    rows_v.at[pl.ds(off + _CH, _CH)], sem),
            )

        def start_rows(k, slot):
            c0, c1 = row_copies(k, slot)
            c0.start()
            c1.start()

        def wait_rows(k, slot):
            c0, c1 = row_copies(k, slot)
            c0.wait()
            c1.wait()

        start_rows(0, 0)

        def task(k, slot):
            @pl.when(k + 1 < _TPW)
            def _():
                start_rows(k + 1, slot ^ 1)
            wait_rows(k, slot)
            rowbase = slot * 2 * _CH
            fyv = fy_v[pl.ds(k * 16, 16)]
            colvs = [x0_v[pl.ds(16 * jc, 16)] for jc in range(_NJC)]
            fxvs = [fx_v[pl.ds(16 * jc, 16)] for jc in range(_NJC)]

            @plsc.parallel_loop(0, _CH, unroll=4)
            def cloop(c):
                rowv0 = zlane + (c + rowbase)
                rowv1 = rowv0 + _CH
                for jc in range(_NJC):
                    colv = colvs[jc]
                    colb = colv + 1
                    a0 = plsc.load_gather(rows_v, [rowv0, colv])
                    b0 = plsc.load_gather(rows_v, [rowv0, colb])
                    a1 = plsc.load_gather(rows_v, [rowv1, colv])
                    b1 = plsc.load_gather(rows_v, [rowv1, colb])
                    fxv = fxvs[jc]
                    t0 = a0 + fxv * (b0 - a0)
                    t1 = a1 + fxv * (b1 - a1)
                    out_v[c, pl.ds(16 * jc, 16)] = t0 + fyv * (t1 - t0)

            pltpu.sync_copy(out_v, out_hbm.at[base + k])

        def pair(k2, carry):
            task(2 * k2, 0)
            task(2 * k2 + 1, 1)
            return carry

        lax.fori_loop(0, _TPW // 2, pair, 0)

    return run(img3, _X0, _FX, _FY)


def kernel(img):
    img_t = jnp.transpose(img, (0, 1, 3, 2))
    img3 = img_t.reshape(_N * _H, 2, _CH, _W).reshape(_N * _H * 2, _CH, _W)
    out = _sc_resize(img3)
    out_t = out.reshape(_N, _OH, 2, _CH, _OW).reshape(_N, _OH, _C, _OW)
    return jnp.transpose(out_t, (0, 1, 3, 2))


# speedup vs baseline: 1.2114x; 1.2114x over previous
import functools

import numpy as np
import jax
import jax.numpy as jnp
from jax import lax
from jax.experimental import pallas as pl
from jax.experimental.pallas import tpu as pltpu
from jax.experimental.pallas import tpu_sc as plsc

_N, _H, _W, _C = 4, 384, 384, 96
_OH = _OW = 224
_CH = _C // 2
_NTASK = _N * _OH * 2
_NWORK = 32
_TPW = _NTASK // _NWORK
_NJC = _OW // 16


def _axis_tables():
    scale = np.float32(_W / _OW)
    i = np.arange(_OW, dtype=np.float32)
    src = (i + np.float32(0.5)) * scale - np.float32(0.5)
    src = np.maximum(src, np.float32(0.0))
    lo = np.floor(src).astype(np.int32)
    frac = (src - lo.astype(np.float32)).astype(np.float32)
    return lo, frac


def _tables():
    lo, frac = _axis_tables()
    x0_t = lo.copy()
    fx_t = frac.copy()
    t = np.arange(_NTASK, dtype=np.int32)
    i_out = (t // 2) % _OH
    fy_t = np.broadcast_to(frac[i_out][:, None], (_NTASK, 16)).astype(np.float32).reshape(-1).copy()
    return x0_t, fx_t, fy_t


_X0, _FX, _FY = _tables()


def _sc_resize(img3):
    mesh = plsc.VectorSubcoreMesh(core_axis_name="c", subcore_axis_name="s")

    @functools.partial(
        pl.kernel,
        out_type=jax.ShapeDtypeStruct((_NTASK, _CH, _OW), jnp.float32),
        mesh=mesh,
        scratch_types=[
            pltpu.VMEM((_OW,), jnp.int32),
            pltpu.VMEM((_OW,), jnp.float32),
            pltpu.VMEM((_TPW * 16,), jnp.float32),
            pltpu.VMEM((4 * _CH, _W), jnp.float32),
            pltpu.VMEM((2 * _CH, _OW), jnp.float32),
            pltpu.SemaphoreType.DMA,
            pltpu.SemaphoreType.DMA,
            pltpu.SemaphoreType.DMA,
            pltpu.SemaphoreType.DMA,
        ],
        compiler_params=pltpu.CompilerParams(needs_layout_passes=False),
    )
    def run(img_hbm, x0_hbm, fx_hbm, fy_hbm, out_hbm,
            x0_v, fx_v, fy_v, rows_v, out_v, rsem0, rsem1, osem0, osem1):
        wid = lax.axis_index("s") * 2 + lax.axis_index("c")
        base = wid * _TPW
        pltpu.sync_copy(x0_hbm, x0_v)
        pltpu.sync_copy(fx_hbm, fx_v)
        pltpu.sync_copy(fy_hbm.at[pl.ds(base * 16, _TPW * 16)], fy_v)

        rsems = (rsem0, rsem1)
        osems = (osem0, osem1)
        zlane = lax.iota(jnp.int32, 16) * 0

        def out_copy(k, slot):
            return pltpu.make_async_copy(
                out_v.at[pl.ds(slot * _CH, _CH)],
                out_hbm.at[base + k], osems[slot])

        def row_copies(k, slot):
            t = base + k
            ch = lax.rem(t, 2)
            i = lax.rem(lax.div(t, 2), _OH)
            n = lax.div(t, 2 * _OH)
            y0 = lax.div(24 * i + 5, 14)
            s0 = (n * _H + y0) * 2 + ch
            sem = rsems[slot]
            off = slot * 2 * _CH
            return (
                pltpu.make_async_copy(
                    img_hbm.at[s0], rows_v.at[pl.ds(off, _CH)], sem),
                pltpu.make_async_copy(
                    img_hbm.at[s0 + 2],
                    rows_v.at[pl.ds(off + _CH, _CH)], sem),
            )

        def start_rows(k, slot):
            c0, c1 = row_copies(k, slot)
            c0.start()
            c1.start()

        def wait_rows(k, slot):
            c0, c1 = row_copies(k, slot)
            c0.wait()
            c1.wait()

        start_rows(0, 0)

        def task(k2, k, slot):
            @pl.when(k + 1 < _TPW)
            def _():
                start_rows(k + 1, slot ^ 1)
            wait_rows(k, slot)

            @pl.when(k2 >= 1)
            def _():
                out_copy(k - 2, slot).wait()

            rowbase = slot * 2 * _CH
            obase = slot * _CH
            fyv = fy_v[pl.ds(k * 16, 16)]
            colvs = [x0_v[pl.ds(16 * jc, 16)] for jc in range(_NJC)]
            fxvs = [fx_v[pl.ds(16 * jc, 16)] for jc in range(_NJC)]

            @plsc.parallel_loop(0, _CH, unroll=2)
            def cloop(c):
                rowv0 = zlane + (c + rowbase)
                rowv1 = rowv0 + _CH
                for jc in range(_NJC):
                    colv = colvs[jc]
                    colb = colv + 1
                    a0 = plsc.load_gather(rows_v, [rowv0, colv])
                    b0 = plsc.load_gather(rows_v, [rowv0, colb])
                    a1 = plsc.load_gather(rows_v, [rowv1, colv])
                    b1 = plsc.load_gather(rows_v, [rowv1, colb])
                    fxv = fxvs[jc]
                    t0 = a0 + fxv * (b0 - a0)
                    t1 = a1 + fxv * (b1 - a1)
                    out_v[obase + c, pl.ds(16 * jc, 16)] = t0 + fyv * (t1 - t0)

            out_copy(k, slot).start()

        def pair(k2, carry):
            task(k2, 2 * k2, 0)
            task(k2, 2 * k2 + 1, 1)
            return carry

        lax.fori_loop(0, _TPW // 2, pair, 0)
        out_copy(_TPW - 2, 0).wait()
        out_copy(_TPW - 1, 1).wait()

    return run(img3, _X0, _FX, _FY)


def kernel(img):
    img_t = jnp.transpose(img, (0, 1, 3, 2))
    img3 = img_t.reshape(_N * _H, 2, _CH, _W).reshape(_N * _H * 2, _CH, _W)
    out = _sc_resize(img3)
    out_t = out.reshape(_N, _OH, 2, _CH, _OW).reshape(_N, _OH, _C, _OW)
    return jnp.transpose(out_t, (0, 1, 3, 2))
